# baseline (speedup 1.0000x reference)
import jax
import jax.numpy as jnp
from jax.experimental import pallas as pl
from jax.experimental.pallas import tpu as pltpu

TBM = 256
NBUF = 2


def _body(hbm_ref, w_ref, pi_ref, b_ref, out_ref, buf, sems):
    tc = pl.program_id(0)
    nst = pl.num_programs(0)
    nb = buf.shape[1]

    def start(step, slot):
        for bi in range(nb):
            pltpu.make_async_copy(
                hbm_ref.at[bi, pl.ds(step * TBM, TBM), :],
                buf.at[slot, bi],
                sems.at[slot, bi],
            ).start()

    def wait(slot):
        for bi in range(nb):
            pltpu.make_async_copy(
                hbm_ref.at[0, pl.ds(0, TBM), :],
                buf.at[slot, bi],
                sems.at[slot, bi],
            ).wait()

    @pl.when(tc == 0)
    def _():
        start(0, 0)

    @pl.when(tc + 1 < nst)
    def _():
        start(tc + 1, (tc + 1) % NBUF)

    slot = tc % NBUF
    wait(slot)
    w = w_ref[...]
    scale = pi_ref[...]
    bias = b_ref[...]
    for bi in range(nb):
        scores = jax.lax.dot_general(
            buf[slot, bi], w, (((1,), (1,)), ((), ())),
            preferred_element_type=jnp.float32,
        )
        out_ref[:, bi, :] = (scores + bias) * scale


def kernel(seq, pi, W, b):
    B, T, D = seq.shape
    E = W.shape[0]
    full = pl.pallas_call(
        _body,
        grid=(T // TBM,),
        in_specs=[
            pl.BlockSpec(memory_space=pltpu.MemorySpace.HBM),
            pl.BlockSpec((E, D), lambda tc: (0, 0)),
            pl.BlockSpec((1, E), lambda tc: (0, 0)),
            pl.BlockSpec((1, E), lambda tc: (0, 0)),
        ],
        out_specs=pl.BlockSpec((TBM, B, E), lambda tc: (tc, 0, 0)),
        out_shape=jax.ShapeDtypeStruct((T, B, E), jnp.float32),
        scratch_shapes=[
            pltpu.VMEM((NBUF, B, TBM, D), jnp.float32),
            pltpu.SemaphoreType.DMA((NBUF, B)),
        ],
        compiler_params=pltpu.CompilerParams(
            dimension_semantics=("arbitrary",),
        ),
    )(seq, W, pi.reshape(1, E), b.reshape(1, E))
    return full[1:]
